# baseline (speedup 1.0000x reference)
import jax
import jax.numpy as jnp
from jax.experimental import pallas as pl
from jax.experimental.pallas import tpu as pltpu

BM = 2048
BN = 2048
BK = 512
NSLICE = 4
BNS = BN // NSLICE


def _matmul_zero_diag_kernel(x_ref, w_ref, o_ref):
    nj = pl.program_id(1)
    kk = pl.program_id(2)

    x = x_ref[...].astype(jnp.bfloat16)
    for s in range(NSLICE):
        w = w_ref[:, s * BNS:(s + 1) * BNS]
        row_ids = kk * BK + jax.lax.broadcasted_iota(jnp.int32, (BK, BNS), 0)
        col_ids = (nj * BN + s * BNS
                   + jax.lax.broadcasted_iota(jnp.int32, (BK, BNS), 1))
        w = jnp.where(row_ids == col_ids, 0.0, w).astype(jnp.bfloat16)
        acc = jnp.dot(x, w, preferred_element_type=jnp.float32)

        @pl.when(kk == 0)
        def _():
            o_ref[:, s * BNS:(s + 1) * BNS] = acc

        @pl.when(kk != 0)
        def _():
            o_ref[:, s * BNS:(s + 1) * BNS] += acc


@jax.jit
def kernel(x, W):
    M, K = x.shape
    _, N = W.shape
    grid = (M // BM, N // BN, K // BK)
    return pl.pallas_call(
        _matmul_zero_diag_kernel,
        grid=grid,
        in_specs=[
            pl.BlockSpec((BM, BK), lambda mi, nj, kk: (mi, kk)),
            pl.BlockSpec((BK, BN), lambda mi, nj, kk: (kk, nj)),
        ],
        out_specs=pl.BlockSpec((BM, BN), lambda mi, nj, kk: (mi, nj)),
        out_shape=jax.ShapeDtypeStruct((M, N), jnp.float32),
        compiler_params=pltpu.CompilerParams(
            dimension_semantics=("parallel", "parallel", "arbitrary"),
            vmem_limit_bytes=112 * 1024 * 1024,
        ),
    )(x, W)


# speedup vs baseline: 1.2416x; 1.2416x over previous
import jax
import jax.numpy as jnp
from jax.experimental import pallas as pl
from jax.experimental.pallas import tpu as pltpu

BM = 2048
BN = 2048
BK = 512


def _matmul_zero_diag_kernel(x_ref, w_ref, o_ref):
    nj = pl.program_id(1)
    kk = pl.program_id(2)

    w = w_ref[...]
    row_ids = kk * BK + jax.lax.broadcasted_iota(jnp.int32, (BK, BN), 0)
    col_ids = nj * BN + jax.lax.broadcasted_iota(jnp.int32, (BK, BN), 1)
    w = jnp.where(row_ids == col_ids, 0.0, w).astype(jnp.bfloat16)
    x = x_ref[...].astype(jnp.bfloat16)

    @pl.when(kk == 0)
    def _():
        o_ref[...] = jnp.dot(x, w, preferred_element_type=jnp.float32)

    @pl.when(kk != 0)
    def _():
        o_ref[...] += jnp.dot(x, w, preferred_element_type=jnp.float32)


@jax.jit
def kernel(x, W):
    M, K = x.shape
    _, N = W.shape
    grid = (M // BM, N // BN, K // BK)
    return pl.pallas_call(
        _matmul_zero_diag_kernel,
        grid=grid,
        in_specs=[
            pl.BlockSpec((BM, BK), lambda mi, nj, kk: (mi, kk)),
            pl.BlockSpec((BK, BN), lambda mi, nj, kk: (kk, nj)),
        ],
        out_specs=pl.BlockSpec((BM, BN), lambda mi, nj, kk: (mi, nj)),
        out_shape=jax.ShapeDtypeStruct((M, N), jnp.float32),
        compiler_params=pltpu.CompilerParams(
            dimension_semantics=("parallel", "parallel", "arbitrary"),
            vmem_limit_bytes=112 * 1024 * 1024,
        ),
    )(x, W)
